# baseline (speedup 1.0000x reference)
import functools
import math

import numpy as np
import jax
import jax.numpy as jnp
from jax import lax
from jax.experimental import pallas as pl
from jax.experimental.pallas import tpu as pltpu
from jax.experimental.pallas import tpu_sc as plsc

_NUM_EMBED = 1000000
_D = 64
_L = 200
_B = 4096
_NW = 32
_FLAT = _B * _L
_ROWS_W = _FLAT // _NW
_CH = 400
_NCH = _ROWS_W // _CH
_SUBS = ((0, 128), (128, 128), (256, 128), (384, 16))


def _pe_pattern() -> np.ndarray:
    position = np.arange(0, _L, dtype=np.float32)[:, None]
    div_term = np.exp(
        np.arange(0, _D, 2, dtype=np.float32) * (-math.log(10000.0) / _D))
    pe = np.zeros((_L, _D), dtype=np.float32)
    pe[:, 0::2] = np.sin(position * div_term)
    pe[:, 1::2] = np.cos(position * div_term)
    return np.tile(pe, (_CH // _L, 1))


def kernel(x, table):
    x_flat = x.reshape(_FLAT)
    pe_rep = jnp.asarray(_pe_pattern())

    mesh = plsc.VectorSubcoreMesh(core_axis_name="c", subcore_axis_name="s")

    @functools.partial(
        pl.kernel,
        mesh=mesh,
        compiler_params=pltpu.CompilerParams(use_tc_tiling_on_sc=False),
        out_type=jax.ShapeDtypeStruct((_FLAT, _D), jnp.float32),
        scratch_types=[
            pltpu.VMEM((_CH,), jnp.int32),
            pltpu.VMEM((_CH, _D), jnp.float32),
            pltpu.VMEM((_CH, _D), jnp.float32),
            pltpu.SemaphoreType.DMA,
        ],
    )
    def run(x_hbm, pe_hbm, table_hbm, out_hbm, idx_v, buf, pe_v, sem):
        nc = 2
        wid = lax.axis_index("s") * nc + lax.axis_index("c")
        base = wid * _ROWS_W
        pltpu.sync_copy(pe_hbm, pe_v)

        def chunk(c, carry):
            row0 = base + c * _CH
            pltpu.sync_copy(x_hbm.at[pl.ds(row0, _CH)], idx_v)
            copies = []
            for off, n in _SUBS:
                copies.append(
                    pltpu.async_copy(
                        table_hbm.at[idx_v.at[pl.ds(off, n)]],
                        buf.at[pl.ds(off, n)],
                        sem,
                    ))
            for cp in copies:
                cp.wait()

            def add_row(r, inner):
                for cc in range(0, _D, 16):
                    plsc.addupdate(
                        buf.at[r, pl.ds(cc, 16)], pe_v[r, pl.ds(cc, 16)])
                return inner

            lax.fori_loop(0, _CH, add_row, 0)
            pltpu.sync_copy(buf, out_hbm.at[pl.ds(row0, _CH)])
            return carry

        lax.fori_loop(0, _NCH, chunk, 0)

    out = run(x_flat, pe_rep, table)
    return out.reshape(_B, _L, _D)


# speedup vs baseline: 1.1527x; 1.1527x over previous
import functools
import math

import numpy as np
import jax
import jax.numpy as jnp
from jax import lax
from jax.experimental import pallas as pl
from jax.experimental.pallas import tpu as pltpu
from jax.experimental.pallas import tpu_sc as plsc

_NUM_EMBED = 1000000
_D = 64
_L = 200
_B = 4096
_NW = 32
_FLAT = _B * _L
_ROWS_W = _FLAT // _NW
_CH = _L
_NCH = _ROWS_W // _CH
_NBUF = 4
_LA = 2
_SUBS = ((0, 128), (128, 72))


def _pe_pattern() -> np.ndarray:
    position = np.arange(0, _L, dtype=np.float32)[:, None]
    div_term = np.exp(
        np.arange(0, _D, 2, dtype=np.float32) * (-math.log(10000.0) / _D))
    pe = np.zeros((_L, _D), dtype=np.float32)
    pe[:, 0::2] = np.sin(position * div_term)
    pe[:, 1::2] = np.cos(position * div_term)
    return pe


def kernel(x, table):
    x_flat = x.reshape(_FLAT)
    pe_rep = jnp.asarray(_pe_pattern())

    mesh = plsc.VectorSubcoreMesh(core_axis_name="c", subcore_axis_name="s")

    @functools.partial(
        pl.kernel,
        mesh=mesh,
        compiler_params=pltpu.CompilerParams(use_tc_tiling_on_sc=False),
        out_type=jax.ShapeDtypeStruct((_FLAT, _D), jnp.float32),
        scratch_types=[
            pltpu.VMEM((_ROWS_W,), jnp.int32),
            pltpu.VMEM((_NBUF, _CH, _D), jnp.float32),
            pltpu.VMEM((_CH, _D), jnp.float32),
            [pltpu.SemaphoreType.DMA] * _NBUF,
            [pltpu.SemaphoreType.DMA] * _NBUF,
        ],
    )
    def run(x_hbm, pe_hbm, table_hbm, out_hbm, idx_v, bufs, pe_v, gsems, wsems):
        nc = 2
        wid = lax.axis_index("s") * nc + lax.axis_index("c")
        base = wid * _ROWS_W
        pltpu.sync_copy(x_hbm.at[pl.ds(base, _ROWS_W)], idx_v)
        pltpu.sync_copy(pe_hbm, pe_v)

        def fire(c, b):
            for off, n in _SUBS:
                pltpu.async_copy(
                    table_hbm.at[idx_v.at[pl.ds(c * _CH + off, n)]],
                    bufs.at[b, pl.ds(off, n)],
                    gsems[b],
                )

        def wait_gathers(b):
            for off, n in _SUBS:
                pltpu.make_async_copy(
                    table_hbm.at[idx_v.at[pl.ds(off, n)]],
                    bufs.at[b, pl.ds(off, n)],
                    gsems[b],
                ).wait()

        def wait_writeback(b):
            pltpu.make_async_copy(
                bufs.at[b], out_hbm.at[pl.ds(0, _CH)], wsems[b]).wait()

        for c0 in range(_LA):
            fire(c0, c0 % _NBUF)

        @pl.loop(0, _NCH, step=_NBUF)
        def chunk_body(c):
            for b in range(_NBUF):
                cc = c + b
                nb = (b + _LA) % _NBUF

                @pl.when(cc + _LA < _NCH)
                def _():
                    @pl.when(cc >= _NBUF - _LA)
                    def _():
                        wait_writeback(nb)

                    fire(cc + _LA, nb)

                wait_gathers(b)

                @plsc.parallel_loop(0, _CH, unroll=4)
                def add_row(r):
                    for col in range(0, _D, 16):
                        plsc.addupdate(
                            bufs.at[b, r, pl.ds(col, 16)],
                            pe_v[r, pl.ds(col, 16)])

                pltpu.async_copy(
                    bufs.at[b],
                    out_hbm.at[pl.ds(base + cc * _CH, _CH)],
                    wsems[b])

        for b in range(_NBUF):
            wait_writeback(b)

    out = run(x_flat, pe_rep, table)
    return out.reshape(_B, _L, _D)
